# baseline (speedup 1.0000x reference)
import jax
import jax.numpy as jnp
from jax import lax
from jax.experimental import pallas as pl
from jax.experimental.pallas import tpu as pltpu
from jax.experimental.pallas import tpu_sc as plsc

_D = 64
_SCALE = 8.0
_NC, _NS = 2, 16
_NW = _NC * _NS
_CHUNK = 128
_BLK = 128
_NB = 4


def _mesh():
    return plsc.VectorSubcoreMesh(
        core_axis_name="c", subcore_axis_name="s",
        num_cores=_NC, num_subcores=_NS,
    )


def _repack_body(table_hbm, dup_hbm,
                 va0, va1, va2, va3, vb0, vb1, vb2, vb3,
                 rs0, rs1, rs2, rs3, ws0, ws1, ws2, ws3):
    c = lax.axis_index("c")
    s = lax.axis_index("s")
    wid = s * _NC + c
    v_rows = table_hbm.shape[0]
    n_full = v_rows // _BLK
    tail = v_rows - n_full * _BLK
    n_t = (n_full + _NW - 1) // _NW
    n_t = ((n_t + _NB - 1) // _NB) * _NB

    vas, vbs = (va0, va1, va2, va3), (vb0, vb1, vb2, vb3)
    rss, wss = (rs0, rs1, rs2, rs3), (ws0, ws1, ws2, ws3)

    def blk_of(t):
        return jnp.minimum(wid + t * _NW, n_full - 1)

    def start_read(t, b):
        pltpu.async_copy(table_hbm.at[pl.ds(blk_of(t) * _BLK, _BLK)], vas[b], rss[b])

    def wait_read(b):
        pltpu.make_async_copy(table_hbm.at[pl.ds(0, _BLK)], vas[b], rss[b]).wait()

    def start_write(t, b):
        pltpu.async_copy(vbs[b], dup_hbm.at[pl.ds(blk_of(t) * _BLK, _BLK)], wss[b])

    def wait_write(b):
        pltpu.make_async_copy(vbs[b], dup_hbm.at[pl.ds(0, _BLK)], wss[b]).wait()

    def repack(b):
        src, dst = vas[b], vbs[b]

        @pl.loop(0, _BLK, step=2, unroll=2)
        def _r(r):
            vals = [src[r + k, pl.ds(16 * j, 16)]
                    for k in range(2) for j in range(_D // 16)]
            for k in range(2):
                for j in range(_D // 16):
                    dst[r + k, pl.ds(16 * j, 16)] = vals[k * 4 + j] * _SCALE

    for b in range(_NB):
        start_read(b, b)

    for b in range(_NB):
        wait_read(b)
        repack(b)
        start_read(_NB + b, b)
        start_write(b, b)

    @pl.loop(_NB, n_t - _NB, step=_NB)
    def _steady(t):
        for b in range(_NB):
            tt = t + b
            wait_read(b)
            wait_write(b)
            repack(b)
            start_read(tt + _NB, b)
            start_write(tt, b)

    for b in range(_NB):
        tt = n_t - _NB + b
        wait_read(b)
        wait_write(b)
        repack(b)
        start_write(tt, b)

    for b in range(_NB):
        wait_write(b)

    if tail:
        @pl.when(wid == 0)
        def _():
            base = n_full * _BLK
            pltpu.sync_copy(table_hbm.at[pl.ds(base, tail)],
                            va0.at[pl.ds(0, tail)])

            @pl.loop(0, tail)
            def _r(r):
                for j in range(_D // 16):
                    sl = pl.ds(16 * j, 16)
                    vb0[r, sl] = va0[r, sl] * _SCALE

            pltpu.sync_copy(vb0.at[pl.ds(0, tail)],
                            dup_hbm.at[pl.ds(base, tail)])


def _gather_body(idx_hbm, dup_hbm, out_hbm,
                 idx_v, in0, in1, o0, o1, gs0, gs1, os0, os1):
    c = lax.axis_index("c")
    s = lax.axis_index("s")
    wid = s * _NC + c
    n_chunks = idx_hbm.shape[1]
    rows_per_w = n_chunks * _CHUNK

    ins, outs, gss, oss = (in0, in1), (o0, o1), (gs0, gs1), (os0, os1)

    def start_gather(ch, b):
        pltpu.async_copy(dup_hbm.at[idx_v.at[ch]], ins[b], gss[b])

    def wait_gather(b):
        pltpu.make_async_copy(dup_hbm.at[idx_v.at[0]], ins[b], gss[b]).wait()

    def start_out(ch, b):
        base = wid * rows_per_w + ch * _CHUNK
        pltpu.async_copy(outs[b], out_hbm.at[pl.ds(base, _CHUNK)], oss[b])

    def wait_out(b):
        pltpu.make_async_copy(outs[b], out_hbm.at[pl.ds(0, _CHUNK)], oss[b]).wait()

    def extract(b):
        src, dst = ins[b], outs[b]

        @pl.loop(0, _CHUNK, unroll=2)
        def _r(r):
            vals = [src[r, pl.ds(16 * j, 16)] for j in range(_D // 16)]
            for j in range(_D // 16):
                dst[r, pl.ds(16 * j, 16)] = vals[j]

    pltpu.sync_copy(idx_hbm.at[wid], idx_v)

    start_gather(0, 0)
    start_gather(1, 1)

    for b in range(2):
        wait_gather(b)
        extract(b)
        start_gather(2 + b, b)
        start_out(b, b)

    @pl.loop(2, n_chunks - 2, step=2)
    def _steady(i):
        for b in range(2):
            ch = i + b
            wait_gather(b)
            wait_out(b)
            extract(b)
            start_gather(ch + 2, b)
            start_out(ch, b)

    for b in range(2):
        ch = n_chunks - 2 + b
        wait_gather(b)
        wait_out(b)
        extract(b)
        start_out(ch, b)

    for b in range(2):
        wait_out(b)


def kernel(x, table):
    b0, b1 = x.shape
    total = b0 * b1
    n_chunks = total // (_NW * _CHUNK)
    assert n_chunks * _NW * _CHUNK == total and n_chunks >= 4 and n_chunks % 2 == 0
    v_rows = table.shape[0]

    x3 = x.astype(jnp.int32).reshape(_NW, n_chunks, _CHUNK)

    repack = pl.kernel(
        _repack_body,
        out_type=jax.ShapeDtypeStruct((v_rows, 2 * _D), jnp.float32),
        mesh=_mesh(),
        compiler_params=pltpu.CompilerParams(needs_layout_passes=False),
        scratch_types=(
            [pltpu.VMEM((_BLK, _D), jnp.float32)] * 4
            + [pltpu.VMEM((_BLK, 2 * _D), jnp.float32)] * 4
            + [pltpu.SemaphoreType.DMA] * 8
        ),
    )
    gather = pl.kernel(
        _gather_body,
        out_type=jax.ShapeDtypeStruct((total, _D), jnp.float32),
        mesh=_mesh(),
        compiler_params=pltpu.CompilerParams(needs_layout_passes=False),
        scratch_types=[
            pltpu.VMEM((n_chunks, _CHUNK), jnp.int32),
            pltpu.VMEM((_CHUNK, 2 * _D), jnp.float32),
            pltpu.VMEM((_CHUNK, 2 * _D), jnp.float32),
            pltpu.VMEM((_CHUNK, _D), jnp.float32),
            pltpu.VMEM((_CHUNK, _D), jnp.float32),
            pltpu.SemaphoreType.DMA,
            pltpu.SemaphoreType.DMA,
            pltpu.SemaphoreType.DMA,
            pltpu.SemaphoreType.DMA,
        ],
    )

    dup = repack(table)
    out = gather(x3, dup)
    return out.reshape(b0, b1, _D)


# speedup vs baseline: 1.0046x; 1.0046x over previous
import jax
import jax.numpy as jnp
from jax import lax
from jax.experimental import pallas as pl
from jax.experimental.pallas import tpu as pltpu
from jax.experimental.pallas import tpu_sc as plsc

_D = 64
_SCALE = 8.0
_NC, _NS = 2, 16
_NW = _NC * _NS
_CHUNK = 128
_BLK = 256


def _mesh():
    return plsc.VectorSubcoreMesh(
        core_axis_name="c", subcore_axis_name="s",
        num_cores=_NC, num_subcores=_NS,
    )


def _repack_body(table_hbm, dup_hbm, va0, va1, vb0, vb1, rs0, rs1, ws0, ws1):
    c = lax.axis_index("c")
    s = lax.axis_index("s")
    wid = s * _NC + c
    v_rows = table_hbm.shape[0]
    n_full = v_rows // _BLK
    tail = v_rows - n_full * _BLK
    n_t = (n_full + _NW - 1) // _NW
    n_t = ((n_t + 1) // 2) * 2

    vas, vbs, rss, wss = (va0, va1), (vb0, vb1), (rs0, rs1), (ws0, ws1)

    def blk_of(t):
        return jnp.minimum(wid + t * _NW, n_full - 1)

    def start_read(t, b):
        pltpu.async_copy(table_hbm.at[pl.ds(blk_of(t) * _BLK, _BLK)], vas[b], rss[b])

    def wait_read(b):
        pltpu.make_async_copy(table_hbm.at[pl.ds(0, _BLK)], vas[b], rss[b]).wait()

    def start_write(t, b):
        pltpu.async_copy(vbs[b], dup_hbm.at[pl.ds(blk_of(t) * _BLK, _BLK)], wss[b])

    def wait_write(b):
        pltpu.make_async_copy(vbs[b], dup_hbm.at[pl.ds(0, _BLK)], wss[b]).wait()

    def repack(b):
        src, dst = vas[b], vbs[b]

        @pl.loop(0, _BLK, step=2, unroll=2)
        def _r(r):
            vals = [src[r + k, pl.ds(16 * j, 16)]
                    for k in range(2) for j in range(_D // 16)]
            for k in range(2):
                for j in range(_D // 16):
                    dst[r + k, pl.ds(16 * j, 16)] = vals[k * 4 + j] * _SCALE

    start_read(0, 0)
    start_read(1, 1)

    for b in range(2):
        wait_read(b)
        repack(b)
        start_read(2 + b, b)
        start_write(b, b)

    @pl.loop(2, n_t - 2, step=2)
    def _steady(t):
        for b in range(2):
            tt = t + b
            wait_read(b)
            wait_write(b)
            repack(b)
            start_read(tt + 2, b)
            start_write(tt, b)

    for b in range(2):
        tt = n_t - 2 + b
        wait_read(b)
        wait_write(b)
        repack(b)
        start_write(tt, b)

    for b in range(2):
        wait_write(b)

    if tail:
        @pl.when(wid == 0)
        def _():
            base = n_full * _BLK
            pltpu.sync_copy(table_hbm.at[pl.ds(base, tail)],
                            va0.at[pl.ds(0, tail)])

            @pl.loop(0, tail)
            def _r(r):
                for j in range(_D // 16):
                    sl = pl.ds(16 * j, 16)
                    vb0[r, sl] = va0[r, sl] * _SCALE

            pltpu.sync_copy(vb0.at[pl.ds(0, tail)],
                            dup_hbm.at[pl.ds(base, tail)])


def _gather_body(idx_hbm, dup_hbm, out_hbm,
                 idx_v, in0, in1, o0, o1, gs0, gs1, os0, os1):
    c = lax.axis_index("c")
    s = lax.axis_index("s")
    wid = s * _NC + c
    n_chunks = idx_hbm.shape[1]
    rows_per_w = n_chunks * _CHUNK

    ins, outs, gss, oss = (in0, in1), (o0, o1), (gs0, gs1), (os0, os1)

    def start_gather(ch, b):
        pltpu.async_copy(dup_hbm.at[idx_v.at[ch]], ins[b], gss[b])

    def wait_gather(b):
        pltpu.make_async_copy(dup_hbm.at[idx_v.at[0]], ins[b], gss[b]).wait()

    def start_out(ch, b):
        base = wid * rows_per_w + ch * _CHUNK
        pltpu.async_copy(outs[b], out_hbm.at[pl.ds(base, _CHUNK)], oss[b])

    def wait_out(b):
        pltpu.make_async_copy(outs[b], out_hbm.at[pl.ds(0, _CHUNK)], oss[b]).wait()

    def extract(b):
        src, dst = ins[b], outs[b]

        @pl.loop(0, _CHUNK, unroll=2)
        def _r(r):
            vals = [src[r, pl.ds(16 * j, 16)] for j in range(_D // 16)]
            for j in range(_D // 16):
                dst[r, pl.ds(16 * j, 16)] = vals[j]

    pltpu.sync_copy(idx_hbm.at[wid], idx_v)

    start_gather(0, 0)
    start_gather(1, 1)

    for b in range(2):
        wait_gather(b)
        extract(b)
        start_gather(2 + b, b)
        start_out(b, b)

    @pl.loop(2, n_chunks - 2, step=2)
    def _steady(i):
        for b in range(2):
            ch = i + b
            wait_gather(b)
            wait_out(b)
            extract(b)
            start_gather(ch + 2, b)
            start_out(ch, b)

    for b in range(2):
        ch = n_chunks - 2 + b
        wait_gather(b)
        wait_out(b)
        extract(b)
        start_out(ch, b)

    for b in range(2):
        wait_out(b)


def kernel(x, table):
    b0, b1 = x.shape
    total = b0 * b1
    n_chunks = total // (_NW * _CHUNK)
    assert n_chunks * _NW * _CHUNK == total and n_chunks >= 4 and n_chunks % 2 == 0
    v_rows = table.shape[0]

    x3 = x.astype(jnp.int32).reshape(_NW, n_chunks, _CHUNK)

    repack = pl.kernel(
        _repack_body,
        out_type=jax.ShapeDtypeStruct((v_rows, 2 * _D), jnp.float32),
        mesh=_mesh(),
        compiler_params=pltpu.CompilerParams(needs_layout_passes=False),
        scratch_types=[
            pltpu.VMEM((_BLK, _D), jnp.float32),
            pltpu.VMEM((_BLK, _D), jnp.float32),
            pltpu.VMEM((_BLK, 2 * _D), jnp.float32),
            pltpu.VMEM((_BLK, 2 * _D), jnp.float32),
            pltpu.SemaphoreType.DMA,
            pltpu.SemaphoreType.DMA,
            pltpu.SemaphoreType.DMA,
            pltpu.SemaphoreType.DMA,
        ],
    )
    gather = pl.kernel(
        _gather_body,
        out_type=jax.ShapeDtypeStruct((total, _D), jnp.float32),
        mesh=_mesh(),
        compiler_params=pltpu.CompilerParams(needs_layout_passes=False),
        scratch_types=[
            pltpu.VMEM((n_chunks, _CHUNK), jnp.int32),
            pltpu.VMEM((_CHUNK, 2 * _D), jnp.float32),
            pltpu.VMEM((_CHUNK, 2 * _D), jnp.float32),
            pltpu.VMEM((_CHUNK, _D), jnp.float32),
            pltpu.VMEM((_CHUNK, _D), jnp.float32),
            pltpu.SemaphoreType.DMA,
            pltpu.SemaphoreType.DMA,
            pltpu.SemaphoreType.DMA,
            pltpu.SemaphoreType.DMA,
        ],
    )

    dup = repack(table)
    out = gather(x3, dup)
    return out.reshape(b0, b1, _D)
